# baseline (speedup 1.0000x reference)
import functools

import jax
import jax.numpy as jnp
from jax import lax
from jax.experimental import pallas as pl
from jax.experimental.pallas import tpu as pltpu
from jax.experimental.pallas import tpu_sc as plsc

NUM_CLASSES = 1000000
D = 16
B = 16384

_info = plsc.get_sparse_core_info()
_NC = _info.num_cores
_NS = _info.num_subcores
_L = _info.num_lanes
NW = _NC * _NS
BPW = B // NW
CHUNK = 128
NCHUNK = BPW // CHUNK


def _sc_body(idx_hbm, ow_hbm, eps_hbm, mean_hbm, logstd_hbm, out_hbm,
             idx_v, mean_v, logstd_v, eps_v, ow_v, out_v, sem):
    wid = lax.axis_index("s") * _NC + lax.axis_index("c")
    pltpu.sync_copy(idx_hbm.at[wid], idx_v)
    copies = []
    for j in range(NCHUNK):
        rows = pl.ds(j * CHUNK, CHUNK)
        copies.append(pltpu.async_copy(mean_hbm.at[idx_v.at[j]],
                                       mean_v.at[rows], sem))
        copies.append(pltpu.async_copy(logstd_hbm.at[idx_v.at[j]],
                                       logstd_v.at[rows], sem))
    pltpu.sync_copy(eps_hbm.at[wid], eps_v)
    pltpu.sync_copy(ow_hbm.at[wid], ow_v)
    for c in copies:
        c.wait()

    def group(g, carry):
        ow16 = ow_v[pl.ds(g * _L, _L)]
        for i in range(_L):
            r = g * _L + i
            w = jnp.broadcast_to(ow16[i], (_L,))
            out_v[r, :] = (mean_v[r, :]
                           + jnp.exp(logstd_v[r, :]) * eps_v[r, :]) * w
        return carry

    lax.fori_loop(0, BPW // _L, group, 0)
    pltpu.sync_copy(out_v, out_hbm.at[wid])


def kernel(indices, session_obs_w, eps, variational_mean, variational_logstd):
    idx3 = indices.reshape(NW, NCHUNK, CHUNK)
    ow2 = session_obs_w.reshape(NW, BPW)
    eps3 = eps.reshape(NW, BPW, D)
    mesh = plsc.VectorSubcoreMesh(core_axis_name="c", subcore_axis_name="s")
    f = functools.partial(
        pl.kernel,
        out_type=jax.ShapeDtypeStruct((NW, BPW, D), jnp.float32),
        mesh=mesh,
        scratch_types=[
            pltpu.VMEM((NCHUNK, CHUNK), jnp.int32),
            pltpu.VMEM((BPW, D), jnp.float32),
            pltpu.VMEM((BPW, D), jnp.float32),
            pltpu.VMEM((BPW, D), jnp.float32),
            pltpu.VMEM((BPW,), jnp.float32),
            pltpu.VMEM((BPW, D), jnp.float32),
            pltpu.SemaphoreType.DMA,
        ],
        compiler_params=pltpu.CompilerParams(use_tc_tiling_on_sc=False),
    )(_sc_body)
    out = f(idx3, ow2, eps3, variational_mean, variational_logstd)
    return out.reshape(B, D)


# speedup vs baseline: 5.4344x; 5.4344x over previous
import functools

import jax
import jax.numpy as jnp
from jax import lax
from jax.experimental import pallas as pl
from jax.experimental.pallas import tpu as pltpu
from jax.experimental.pallas import tpu_sc as plsc

NUM_CLASSES = 1000000
D = 16
B = 16384

_info = plsc.get_sparse_core_info()
_NC = _info.num_cores
_NS = _info.num_subcores
_L = _info.num_lanes
NW = _NC * _NS
BPW = B // NW
NGROUP = BPW // _L


def _sc_body(idx_hbm, ow_hbm, epsT_hbm, meanT_hbm, logstdT_hbm, outT_hbm,
             idx_v, eps_v, ow_v, out_v, *slots_and_sem):
    mslots = slots_and_sem[:_L]
    lslots = slots_and_sem[_L:2 * _L]
    sem = slots_and_sem[2 * _L]
    wid = lax.axis_index("s") * _NC + lax.axis_index("c")
    col0 = wid * BPW
    pltpu.sync_copy(idx_hbm.at[wid], idx_v)
    pltpu.sync_copy(epsT_hbm.at[:, pl.ds(col0, BPW)], eps_v)
    pltpu.sync_copy(ow_hbm.at[pl.ds(col0, BPW)], ow_v)
    iota = lax.iota(jnp.int32, _L)

    def group(g, carry):
        idxvec = idx_v[pl.ds(g * _L, _L)]
        lanes = idxvec & 127
        owvec = ow_v[pl.ds(g * _L, _L)]
        copies = []
        for j in range(_L):
            c = pl.multiple_of((idxvec[j] >> 7) << 7, 128)
            copies.append(pltpu.async_copy(
                meanT_hbm.at[:, pl.ds(c, 128)], mslots[j], sem))
            copies.append(pltpu.async_copy(
                logstdT_hbm.at[:, pl.ds(c, 128)], lslots[j], sem))
        for cp in copies:
            cp.wait()
        for j in range(_L):
            bvec = jnp.broadcast_to(g * _L + j, (_L,))
            lane = jnp.broadcast_to(lanes[j], (_L,))
            w = jnp.broadcast_to(owvec[j], (_L,))
            mcol = plsc.load_gather(mslots[j], [iota, lane])
            lcol = plsc.load_gather(lslots[j], [iota, lane])
            ecol = plsc.load_gather(eps_v, [iota, bvec])
            res = (mcol + jnp.exp(lcol) * ecol) * w
            plsc.store_scatter(out_v, [iota, bvec], res)
        return carry

    lax.fori_loop(0, NGROUP, group, 0)
    pltpu.sync_copy(out_v, outT_hbm.at[:, pl.ds(col0, BPW)])


def kernel(indices, session_obs_w, eps, variational_mean, variational_logstd):
    idx2 = indices.reshape(NW, BPW)
    ow = session_obs_w.reshape(B)
    mesh = plsc.VectorSubcoreMesh(core_axis_name="c", subcore_axis_name="s")
    scratch = [
        pltpu.VMEM((BPW,), jnp.int32),
        pltpu.VMEM((D, BPW), jnp.float32),
        pltpu.VMEM((BPW,), jnp.float32),
        pltpu.VMEM((D, BPW), jnp.float32),
    ]
    scratch += [pltpu.VMEM((D, 128), jnp.float32) for _ in range(2 * _L)]
    scratch += [pltpu.SemaphoreType.DMA]
    f = functools.partial(
        pl.kernel,
        out_type=jax.ShapeDtypeStruct((D, B), jnp.float32),
        mesh=mesh,
        scratch_types=scratch,
        compiler_params=pltpu.CompilerParams(
            use_tc_tiling_on_sc=True, needs_layout_passes=False),
    )(_sc_body)
    outT = f(idx2, ow, eps.T, variational_mean.T, variational_logstd.T)
    return outT.T


# speedup vs baseline: 5.8577x; 1.0779x over previous
import functools

import jax
import jax.numpy as jnp
from jax import lax
from jax.experimental import pallas as pl
from jax.experimental.pallas import tpu as pltpu
from jax.experimental.pallas import tpu_sc as plsc

NUM_CLASSES = 1000000
D = 16
B = 16384

_info = plsc.get_sparse_core_info()
_NC = _info.num_cores
_NS = _info.num_subcores
_L = _info.num_lanes
NW = _NC * _NS
BPW = B // NW
NGROUP = BPW // _L
_H = _L // 2


def _sc_body(idx_hbm, ow_hbm, epsT_hbm, meanT_hbm, logstdT_hbm, outT_hbm,
             idx_v, eps_v, ow_v, out_v, *slots_and_sems):
    ma = slots_and_sems[0:_H]
    la = slots_and_sems[_H:2 * _H]
    mb = slots_and_sems[2 * _H:3 * _H]
    lb = slots_and_sems[3 * _H:4 * _H]
    sema = slots_and_sems[4 * _H]
    semb = slots_and_sems[4 * _H + 1]
    wid = lax.axis_index("s") * _NC + lax.axis_index("c")
    col0 = wid * BPW
    pltpu.sync_copy(idx_hbm.at[wid], idx_v)
    pltpu.sync_copy(epsT_hbm.at[:, pl.ds(col0, BPW)], eps_v)
    pltpu.sync_copy(ow_hbm.at[pl.ds(col0, BPW)], ow_v)
    iota = lax.iota(jnp.int32, _L)

    def fire(idxvec, j0, mslots, lslots, sem):
        for j in range(_H):
            c = pl.multiple_of((idxvec[j0 + j] >> 7) << 7, 128)
            pltpu.async_copy(meanT_hbm.at[:, pl.ds(c, 128)], mslots[j], sem)
            pltpu.async_copy(logstdT_hbm.at[:, pl.ds(c, 128)], lslots[j], sem)

    def drain(mslots, lslots, sem):
        for j in range(_H):
            pltpu.make_async_copy(meanT_hbm.at[:, pl.ds(0, 128)],
                                  mslots[j], sem).wait()
            pltpu.make_async_copy(logstdT_hbm.at[:, pl.ds(0, 128)],
                                  lslots[j], sem).wait()

    def compute(g, idxvec, lanes, owvec, j0, mslots, lslots):
        for j in range(_H):
            bvec = jnp.broadcast_to(g * _L + j0 + j, (_L,))
            lane = jnp.broadcast_to(lanes[j0 + j], (_L,))
            w = jnp.broadcast_to(owvec[j0 + j], (_L,))
            mcol = plsc.load_gather(mslots[j], [iota, lane])
            lcol = plsc.load_gather(lslots[j], [iota, lane])
            ecol = plsc.load_gather(eps_v, [iota, bvec])
            res = (mcol + jnp.exp(lcol) * ecol) * w
            plsc.store_scatter(out_v, [iota, bvec], res)

    idxvec0 = idx_v[pl.ds(0, _L)]
    fire(idxvec0, 0, ma, la, sema)

    def group(g, carry):
        idxvec = idx_v[pl.ds(g * _L, _L)]
        lanes = idxvec & 127
        owvec = ow_v[pl.ds(g * _L, _L)]
        fire(idxvec, _H, mb, lb, semb)
        drain(ma, la, sema)
        compute(g, idxvec, lanes, owvec, 0, ma, la)
        gn = jnp.minimum(g + 1, NGROUP - 1)
        idxnext = idx_v[pl.ds(gn * _L, _L)]
        fire(idxnext, 0, ma, la, sema)
        drain(mb, lb, semb)
        compute(g, idxvec, lanes, owvec, _H, mb, lb)
        return carry

    lax.fori_loop(0, NGROUP, group, 0)
    drain(ma, la, sema)
    pltpu.sync_copy(out_v, outT_hbm.at[:, pl.ds(col0, BPW)])


def kernel(indices, session_obs_w, eps, variational_mean, variational_logstd):
    idx2 = indices.reshape(NW, BPW)
    ow = session_obs_w.reshape(B)
    mesh = plsc.VectorSubcoreMesh(core_axis_name="c", subcore_axis_name="s")
    scratch = [
        pltpu.VMEM((BPW,), jnp.int32),
        pltpu.VMEM((D, BPW), jnp.float32),
        pltpu.VMEM((BPW,), jnp.float32),
        pltpu.VMEM((D, BPW), jnp.float32),
    ]
    scratch += [pltpu.VMEM((D, 128), jnp.float32) for _ in range(4 * _H)]
    scratch += [pltpu.SemaphoreType.DMA, pltpu.SemaphoreType.DMA]
    f = functools.partial(
        pl.kernel,
        out_type=jax.ShapeDtypeStruct((D, B), jnp.float32),
        mesh=mesh,
        scratch_types=scratch,
        compiler_params=pltpu.CompilerParams(
            use_tc_tiling_on_sc=True, needs_layout_passes=False),
    )(_sc_body)
    outT = f(idx2, ow, eps.T, variational_mean.T, variational_logstd.T)
    return outT.T
